# Initial kernel scaffold; baseline (speedup 1.0000x reference)
import jax
import jax.numpy as jnp
from jax.experimental import pallas as pl


def kernel(x, edge_index, Wl1, bl1, Wr1, g1, b1, Wl2, bl2, Wr2, g2, b2, Wl3, bl3, Wr3):
    raise NotImplementedError("write your pallas kernel here")


# speedup vs baseline: 17.3524x; 17.3524x over previous
import functools

import jax
import jax.numpy as jnp
from jax import lax
from jax.experimental import pallas as pl
from jax.experimental.pallas import tpu as pltpu
from jax.experimental.pallas import tpu_sc as plsc

_CH = 128
_NCORE = 2
_NSUB = 16
_NWORK = _NCORE * _NSUB
_GRP = 8


def _pad_up(v, m):
    return (v + m - 1) // m * m


def _make_segsum(n_pad, w, nch):
    rpt = n_pad // _NSUB
    mesh = plsc.VectorSubcoreMesh(core_axis_name="c", subcore_axis_name="s")

    @functools.partial(
        pl.kernel,
        out_type=jax.ShapeDtypeStruct((_NCORE, n_pad, w), jnp.float32),
        mesh=mesh,
        compiler_params=pltpu.CompilerParams(use_tc_tiling_on_sc=False),
        scratch_types=[
            pltpu.VMEM((nch, _CH), jnp.int32),
            pltpu.VMEM((nch, _CH), jnp.int32),
            *[pltpu.VMEM((_CH, w), jnp.float32) for _ in range(_GRP)],
            pltpu.VMEM((rpt, w), jnp.float32),
            pltpu.VMEM_SHARED((n_pad, w), jnp.float32),
            pltpu.SemaphoreType.DMA,
        ],
    )
    def seg(table, src2d, dst2d, out, srcv, dstv, *rest):
        rows = rest[:_GRP]
        cbuf, acc, sem = rest[_GRP], rest[_GRP + 1], rest[_GRP + 2]
        cid = lax.axis_index("c")
        sid = lax.axis_index("s")
        wid = cid * _NSUB + sid

        pltpu.sync_copy(src2d.at[pl.ds(wid * nch, nch)], srcv)
        pltpu.sync_copy(dst2d.at[pl.ds(wid * nch, nch)], dstv)

        zv = jnp.zeros((16,), jnp.float32)

        def zrow(i, carry):
            for j in range(w // 16):
                cbuf[i, pl.ds(j * 16, 16)] = zv
            return carry

        lax.fori_loop(0, rpt, zrow, 0)
        pltpu.sync_copy(cbuf, acc.at[pl.ds(sid * rpt, rpt)])
        plsc.subcore_barrier()

        def chunk_group(p, carry):
            base = p * _GRP
            cps = [
                pltpu.async_copy(table.at[srcv.at[base + j]], rows[j], sem)
                for j in range(_GRP)
            ]
            for cp in cps:
                cp.wait()
            for j in range(_GRP):
                pltpu.sync_copy(rows[j], acc.at[dstv.at[base + j]], add=True)
            return carry

        lax.fori_loop(0, nch // _GRP, chunk_group, 0)
        plsc.subcore_barrier()

        pltpu.sync_copy(acc.at[pl.ds(sid * rpt, rpt)], cbuf)
        pltpu.sync_copy(cbuf, out.at[cid, pl.ds(sid * rpt, rpt)])

    return seg


def kernel(x, edge_index, Wl1, bl1, Wr1, g1, b1, Wl2, bl2, Wr2, g2, b2,
           Wl3, bl3, Wr3):
    n, d_in = x.shape
    e = edge_index.shape[1]
    d_h = Wl1.shape[0]
    d_out = Wl3.shape[0]

    w1 = _pad_up(d_h + 1, 16)
    w2 = _pad_up(d_h, 16)
    w3 = _pad_up(d_out, 16)
    n_pad = _pad_up(n + 1, 16 * _NSUB)
    e_pad = _pad_up(e, _NWORK * _CH * 8)
    nch = e_pad // (_NWORK * _CH)

    src = edge_index[0]
    dst = edge_index[1]
    pad = e_pad - e
    if pad:
        ar = jnp.arange(pad, dtype=jnp.int32)
        src = jnp.concatenate([src, ar % n])
        dst = jnp.concatenate([dst, n + ar % (n_pad - n)])
    src2d = src.reshape(e_pad // _CH, _CH)
    dst2d = dst.reshape(e_pad // _CH, _CH)

    f32 = jnp.float32
    wl1t = jnp.zeros((d_in, w1), f32).at[:, :d_h].set(Wl1.T)
    wr1t = Wr1.T
    wl2t = Wl2.T
    wr2t = Wr2.T
    wl3t = jnp.zeros((d_h, w3), f32).at[:, :d_out].set(Wl3.T)
    wr3t = jnp.zeros((d_h, w3), f32).at[:, :d_out].set(Wr3.T)
    bl1r = bl1.reshape(1, d_h)
    g1r = g1.reshape(1, d_h)
    b1r = b1.reshape(1, d_h)
    bl2r = bl2.reshape(1, d_h)
    g2r = g2.reshape(1, d_h)
    b2r = b2.reshape(1, d_h)
    bl3r = jnp.zeros((w3,), f32).at[:d_out].set(bl3).reshape(1, w3)

    rb = 2000 if n % 2000 == 0 else (1000 if n % 1000 == 0 else 8)
    grid = (n // rb,)
    row_spec = lambda c: pl.BlockSpec((rb, c), lambda i: (i, 0))
    full_spec = lambda r, c: pl.BlockSpec((r, c), lambda i: (0, 0))

    def tc1(x_ref, wl_ref, wr_ref, tab_ref, r_ref):
        xb = x_ref[:]
        t = jnp.dot(xb, wl_ref[:], preferred_element_type=f32)
        ones_col = (lax.broadcasted_iota(jnp.int32, t.shape, 1) == d_h)
        tab_ref[:] = t + ones_col.astype(f32)
        r_ref[:] = jnp.dot(xb, wr_ref[:], preferred_element_type=f32)

    table1, r1 = pl.pallas_call(
        tc1,
        grid=grid,
        in_specs=[row_spec(d_in), full_spec(d_in, w1), full_spec(d_in, d_h)],
        out_specs=[row_spec(w1), row_spec(d_h)],
        out_shape=[jax.ShapeDtypeStruct((n, w1), f32),
                   jax.ShapeDtypeStruct((n, d_h), f32)],
    )(x, wl1t, wr1t)

    acc1 = _make_segsum(n_pad, w1, nch)(table1, src2d, dst2d)

    def tc2(a0_ref, a1_ref, r_ref, bl_ref, g_ref, b_ref, wl_ref, wr_ref,
            tab_ref, rn_ref, inv_ref):
        s = a0_ref[:] + a1_ref[:]
        inv = 1.0 / jnp.clip(s[:, d_h:d_h + 1], 1.0, None)
        pre = s[:, :d_h] * inv + bl_ref[:] + r_ref[:]
        mu = jnp.mean(pre, axis=-1, keepdims=True)
        var = jnp.mean((pre - mu) ** 2, axis=-1, keepdims=True)
        h = (pre - mu) / jnp.sqrt(var + 1e-5) * g_ref[:] + b_ref[:]
        h = jnp.maximum(h, 0.0)
        tab_ref[:] = jnp.dot(h, wl_ref[:], preferred_element_type=f32)
        rn_ref[:] = jnp.dot(h, wr_ref[:], preferred_element_type=f32)
        inv_ref[:] = jnp.broadcast_to(inv, (inv.shape[0], 8))

    table2, r2, invc = pl.pallas_call(
        tc2,
        grid=grid,
        in_specs=[row_spec(w1), row_spec(w1), row_spec(d_h),
                  full_spec(1, d_h), full_spec(1, d_h), full_spec(1, d_h),
                  full_spec(d_h, w2), full_spec(d_h, d_h)],
        out_specs=[row_spec(w2), row_spec(d_h), row_spec(8)],
        out_shape=[jax.ShapeDtypeStruct((n, w2), f32),
                   jax.ShapeDtypeStruct((n, d_h), f32),
                   jax.ShapeDtypeStruct((n, 8), f32)],
    )(acc1[0], acc1[1], r1, bl1r, g1r, b1r, wl2t, wr2t)

    acc2 = _make_segsum(n_pad, w2, nch)(table2, src2d, dst2d)

    def tc3(a0_ref, a1_ref, r_ref, inv_ref, bl_ref, g_ref, b_ref, wl_ref,
            wr_ref, tab_ref, rn_ref):
        s = a0_ref[:] + a1_ref[:]
        pre = s * inv_ref[:][:, :1] + bl_ref[:] + r_ref[:]
        mu = jnp.mean(pre, axis=-1, keepdims=True)
        var = jnp.mean((pre - mu) ** 2, axis=-1, keepdims=True)
        h = (pre - mu) / jnp.sqrt(var + 1e-5) * g_ref[:] + b_ref[:]
        h = jnp.maximum(h, 0.0)
        tab_ref[:] = jnp.dot(h, wl_ref[:], preferred_element_type=f32)
        rn_ref[:] = jnp.dot(h, wr_ref[:], preferred_element_type=f32)

    table3, r3 = pl.pallas_call(
        tc3,
        grid=grid,
        in_specs=[row_spec(w2), row_spec(w2), row_spec(d_h), row_spec(8),
                  full_spec(1, d_h), full_spec(1, d_h), full_spec(1, d_h),
                  full_spec(d_h, w3), full_spec(d_h, w3)],
        out_specs=[row_spec(w3), row_spec(w3)],
        out_shape=[jax.ShapeDtypeStruct((n, w3), f32),
                   jax.ShapeDtypeStruct((n, w3), f32)],
    )(acc2[0], acc2[1], r2, invc, bl2r, g2r, b2r, wl3t, wr3t)

    acc3 = _make_segsum(n_pad, w3, nch)(table3, src2d, dst2d)

    def tc4(a0_ref, a1_ref, r_ref, inv_ref, bl_ref, out_ref):
        s = a0_ref[:] + a1_ref[:]
        out_ref[:] = s * inv_ref[:][:, :1] + bl_ref[:] + r_ref[:]

    out16 = pl.pallas_call(
        tc4,
        grid=grid,
        in_specs=[row_spec(w3), row_spec(w3), row_spec(w3), row_spec(8),
                  full_spec(1, w3)],
        out_specs=row_spec(w3),
        out_shape=jax.ShapeDtypeStruct((n, w3), f32),
    )(acc3[0], acc3[1], r3, invc, bl3r)

    return out16[:, :d_out]


# speedup vs baseline: 20.3764x; 1.1743x over previous
import functools

import jax
import jax.numpy as jnp
from jax import lax
from jax.experimental import pallas as pl
from jax.experimental.pallas import tpu as pltpu
from jax.experimental.pallas import tpu_sc as plsc

_CH = 128
_NCORE = 2
_NSUB = 16
_NWORK = _NCORE * _NSUB


def _pad_up(v, m):
    return (v + m - 1) // m * m


def _make_segsum(n_pad, w, nch):
    rpt = n_pad // _NSUB
    mesh = plsc.VectorSubcoreMesh(core_axis_name="c", subcore_axis_name="s")

    grp = 5 if w > 32 else 8
    assert nch % (2 * grp) == 0
    ngrp = nch // grp

    @functools.partial(
        pl.kernel,
        out_type=jax.ShapeDtypeStruct((_NCORE, n_pad, w), jnp.float32),
        mesh=mesh,
        compiler_params=pltpu.CompilerParams(use_tc_tiling_on_sc=False),
        scratch_types=[
            pltpu.VMEM((nch, _CH), jnp.int32),
            pltpu.VMEM((nch, _CH), jnp.int32),
            *[pltpu.VMEM((_CH, w), jnp.float32) for _ in range(2 * grp)],
            pltpu.VMEM_SHARED((n_pad, w), jnp.float32),
            pltpu.SemaphoreType.DMA,
            pltpu.SemaphoreType.DMA,
        ],
    )
    def seg(table, src2d, dst2d, out, srcv, dstv, *rest):
        bufs = (rest[:grp], rest[grp:2 * grp])
        acc = rest[2 * grp]
        gsem, ssem = rest[2 * grp + 1], rest[2 * grp + 2]
        cid = lax.axis_index("c")
        sid = lax.axis_index("s")
        wid = cid * _NSUB + sid

        pltpu.sync_copy(src2d.at[pl.ds(wid * nch, nch)], srcv)
        pltpu.sync_copy(dst2d.at[pl.ds(wid * nch, nch)], dstv)

        zv = jnp.zeros((16,), jnp.float32)

        def zrow(i, carry):
            for j in range(w // 16):
                bufs[0][0][i, pl.ds(j * 16, 16)] = zv
            return carry

        lax.fori_loop(0, _CH, zrow, 0)

        def zcopy(k, carry):
            pltpu.sync_copy(bufs[0][0],
                            acc.at[pl.ds(sid * rpt + k * _CH, _CH)])
            return carry

        lax.fori_loop(0, rpt // _CH, zcopy, 0)
        plsc.subcore_barrier()

        npair = ngrp // 2

        def wait_gathers(bset):
            for j in range(grp):
                pltpu.make_async_copy(table.at[srcv.at[0]], bset[j],
                                      gsem).wait()

        def fire_gathers(bset, base):
            for j in range(grp):
                pltpu.async_copy(table.at[srcv.at[base + j]], bset[j], gsem)

        def fire_scatters(bset, base):
            for j in range(grp):
                pltpu.async_copy(bset[j], acc.at[dstv.at[base + j]], ssem,
                                 add=True)

        def drain_scatters(bset):
            for j in range(grp):
                pltpu.make_async_copy(bset[j], acc.at[dstv.at[0]],
                                      ssem).wait()

        fire_gathers(bufs[0], 0)

        def pair(q, carry):
            a = 2 * q * grp
            wait_gathers(bufs[0])

            @pl.when(q > 0)
            def _d1():
                drain_scatters(bufs[1])

            fire_gathers(bufs[1], a + grp)
            fire_scatters(bufs[0], a)
            wait_gathers(bufs[1])

            @pl.when(q + 1 < npair)
            def _d0():
                drain_scatters(bufs[0])
                fire_gathers(bufs[0], a + 2 * grp)

            fire_scatters(bufs[1], a + grp)
            return carry

        lax.fori_loop(0, npair, pair, 0)
        drain_scatters(bufs[0])
        drain_scatters(bufs[1])
        plsc.subcore_barrier()

        def cout(k, carry):
            pltpu.sync_copy(acc.at[pl.ds(sid * rpt + k * _CH, _CH)],
                            bufs[0][0])
            pltpu.async_copy(bufs[0][0],
                             out.at[cid, pl.ds(sid * rpt + k * _CH, _CH)],
                             gsem).wait()
            return carry

        lax.fori_loop(0, rpt // _CH, cout, 0)

    return seg


def kernel(x, edge_index, Wl1, bl1, Wr1, g1, b1, Wl2, bl2, Wr2, g2, b2,
           Wl3, bl3, Wr3):
    n, d_in = x.shape
    e = edge_index.shape[1]
    d_h = Wl1.shape[0]
    d_out = Wl3.shape[0]

    w1 = _pad_up(d_h + 1, 16)
    w2 = _pad_up(d_h, 16)
    w3 = _pad_up(d_out, 16)
    n_pad = _pad_up(n + 1, 16 * _NSUB)
    e_pad = _pad_up(e, _NWORK * _CH * 8)
    nch = e_pad // (_NWORK * _CH)

    src = edge_index[0]
    dst = edge_index[1]
    pad = e_pad - e
    if pad:
        ar = jnp.arange(pad, dtype=jnp.int32)
        src = jnp.concatenate([src, ar % n])
        dst = jnp.concatenate([dst, n + ar % (n_pad - n)])
    src2d = src.reshape(e_pad // _CH, _CH)
    dst2d = dst.reshape(e_pad // _CH, _CH)

    f32 = jnp.float32
    wl1t = jnp.zeros((d_in, w1), f32).at[:, :d_h].set(Wl1.T)
    wr1t = Wr1.T
    wl2t = Wl2.T
    wr2t = Wr2.T
    wl3t = jnp.zeros((d_h, w3), f32).at[:, :d_out].set(Wl3.T)
    wr3t = jnp.zeros((d_h, w3), f32).at[:, :d_out].set(Wr3.T)
    bl1r = bl1.reshape(1, d_h)
    g1r = g1.reshape(1, d_h)
    b1r = b1.reshape(1, d_h)
    bl2r = bl2.reshape(1, d_h)
    g2r = g2.reshape(1, d_h)
    b2r = b2.reshape(1, d_h)
    bl3r = jnp.zeros((w3,), f32).at[:d_out].set(bl3).reshape(1, w3)

    rb = 2000 if n % 2000 == 0 else (1000 if n % 1000 == 0 else 8)
    grid = (n // rb,)
    row_spec = lambda c: pl.BlockSpec((rb, c), lambda i: (i, 0))
    full_spec = lambda r, c: pl.BlockSpec((r, c), lambda i: (0, 0))

    def tc1(x_ref, wl_ref, wr_ref, tab_ref, r_ref):
        xb = x_ref[:]
        t = jnp.dot(xb, wl_ref[:], preferred_element_type=f32)
        ones_col = (lax.broadcasted_iota(jnp.int32, t.shape, 1) == d_h)
        tab_ref[:] = t + ones_col.astype(f32)
        r_ref[:] = jnp.dot(xb, wr_ref[:], preferred_element_type=f32)

    table1, r1 = pl.pallas_call(
        tc1,
        grid=grid,
        in_specs=[row_spec(d_in), full_spec(d_in, w1), full_spec(d_in, d_h)],
        out_specs=[row_spec(w1), row_spec(d_h)],
        out_shape=[jax.ShapeDtypeStruct((n, w1), f32),
                   jax.ShapeDtypeStruct((n, d_h), f32)],
    )(x, wl1t, wr1t)

    acc1 = _make_segsum(n_pad, w1, nch)(table1, src2d, dst2d)

    def tc2(a0_ref, a1_ref, r_ref, bl_ref, g_ref, b_ref, wl_ref, wr_ref,
            tab_ref, rn_ref, inv_ref):
        s = a0_ref[:] + a1_ref[:]
        inv = 1.0 / jnp.clip(s[:, d_h:d_h + 1], 1.0, None)
        pre = s[:, :d_h] * inv + bl_ref[:] + r_ref[:]
        mu = jnp.mean(pre, axis=-1, keepdims=True)
        var = jnp.mean((pre - mu) ** 2, axis=-1, keepdims=True)
        h = (pre - mu) / jnp.sqrt(var + 1e-5) * g_ref[:] + b_ref[:]
        h = jnp.maximum(h, 0.0)
        tab_ref[:] = jnp.dot(h, wl_ref[:], preferred_element_type=f32)
        rn_ref[:] = jnp.dot(h, wr_ref[:], preferred_element_type=f32)
        inv_ref[:] = jnp.broadcast_to(inv, (inv.shape[0], 8))

    table2, r2, invc = pl.pallas_call(
        tc2,
        grid=grid,
        in_specs=[row_spec(w1), row_spec(w1), row_spec(d_h),
                  full_spec(1, d_h), full_spec(1, d_h), full_spec(1, d_h),
                  full_spec(d_h, w2), full_spec(d_h, d_h)],
        out_specs=[row_spec(w2), row_spec(d_h), row_spec(8)],
        out_shape=[jax.ShapeDtypeStruct((n, w2), f32),
                   jax.ShapeDtypeStruct((n, d_h), f32),
                   jax.ShapeDtypeStruct((n, 8), f32)],
    )(acc1[0], acc1[1], r1, bl1r, g1r, b1r, wl2t, wr2t)

    acc2 = _make_segsum(n_pad, w2, nch)(table2, src2d, dst2d)

    def tc3(a0_ref, a1_ref, r_ref, inv_ref, bl_ref, g_ref, b_ref, wl_ref,
            wr_ref, tab_ref, rn_ref):
        s = a0_ref[:] + a1_ref[:]
        pre = s * inv_ref[:][:, :1] + bl_ref[:] + r_ref[:]
        mu = jnp.mean(pre, axis=-1, keepdims=True)
        var = jnp.mean((pre - mu) ** 2, axis=-1, keepdims=True)
        h = (pre - mu) / jnp.sqrt(var + 1e-5) * g_ref[:] + b_ref[:]
        h = jnp.maximum(h, 0.0)
        tab_ref[:] = jnp.dot(h, wl_ref[:], preferred_element_type=f32)
        rn_ref[:] = jnp.dot(h, wr_ref[:], preferred_element_type=f32)

    table3, r3 = pl.pallas_call(
        tc3,
        grid=grid,
        in_specs=[row_spec(w2), row_spec(w2), row_spec(d_h), row_spec(8),
                  full_spec(1, d_h), full_spec(1, d_h), full_spec(1, d_h),
                  full_spec(d_h, w3), full_spec(d_h, w3)],
        out_specs=[row_spec(w3), row_spec(w3)],
        out_shape=[jax.ShapeDtypeStruct((n, w3), f32),
                   jax.ShapeDtypeStruct((n, w3), f32)],
    )(acc2[0], acc2[1], r2, invc, bl2r, g2r, b2r, wl3t, wr3t)

    acc3 = _make_segsum(n_pad, w3, nch)(table3, src2d, dst2d)

    def tc4(a0_ref, a1_ref, r_ref, inv_ref, bl_ref, out_ref):
        s = a0_ref[:] + a1_ref[:]
        out_ref[:] = s * inv_ref[:][:, :1] + bl_ref[:] + r_ref[:]

    out16 = pl.pallas_call(
        tc4,
        grid=grid,
        in_specs=[row_spec(w3), row_spec(w3), row_spec(w3), row_spec(8),
                  full_spec(1, w3)],
        out_specs=row_spec(w3),
        out_shape=jax.ShapeDtypeStruct((n, w3), f32),
    )(acc3[0], acc3[1], r3, invc, bl3r)

    return out16[:, :d_out]


# speedup vs baseline: 26.3947x; 1.2954x over previous
import functools

import jax
import jax.numpy as jnp
from jax import lax
from jax.experimental import pallas as pl
from jax.experimental.pallas import tpu as pltpu
from jax.experimental.pallas import tpu_sc as plsc

_CH = 128
_NCORE = 2
_NSUB = 16
_NWORK = _NCORE * _NSUB


def _pad_up(v, m):
    return (v + m - 1) // m * m


def _make_segsum(n_pad, w, nch, with_cnt):
    rpt = n_pad // _NSUB
    mesh = plsc.VectorSubcoreMesh(core_axis_name="c", subcore_axis_name="s")

    grp = 5 if with_cnt else 8
    assert nch % (2 * grp) == 0
    ngrp = nch // grp

    out_shape = jax.ShapeDtypeStruct((_NCORE, n_pad, w), jnp.float32)
    scratch = [
        pltpu.VMEM((nch, _CH), jnp.int32),
        pltpu.VMEM((nch, _CH), jnp.int32),
        *[pltpu.VMEM((_CH, w), jnp.float32) for _ in range(2 * grp)],
        pltpu.VMEM_SHARED((n_pad, w), jnp.float32),
        pltpu.SemaphoreType.DMA,
        pltpu.SemaphoreType.DMA,
    ]
    if with_cnt:
        scratch += [
            pltpu.VMEM((_CH, w), jnp.float32),
            pltpu.VMEM_SHARED((n_pad, w), jnp.float32),
            pltpu.SemaphoreType.DMA,
        ]

    @functools.partial(
        pl.kernel,
        out_type=[out_shape, out_shape] if with_cnt else out_shape,
        mesh=mesh,
        compiler_params=pltpu.CompilerParams(use_tc_tiling_on_sc=False),
        scratch_types=scratch,
    )
    def seg(table, src2d, dst2d, *rest):
        if with_cnt:
            out, cout_hbm = rest[0], rest[1]
            rest = rest[2:]
        else:
            out = rest[0]
            rest = rest[1:]
        srcv, dstv = rest[0], rest[1]
        bufs = (rest[2:2 + grp], rest[2 + grp:2 + 2 * grp])
        acc = rest[2 + 2 * grp]
        gsem, ssem = rest[3 + 2 * grp], rest[4 + 2 * grp]
        if with_cnt:
            obuf, cacc, osem = (rest[5 + 2 * grp], rest[6 + 2 * grp],
                                rest[7 + 2 * grp])
        cid = lax.axis_index("c")
        sid = lax.axis_index("s")
        wid = cid * _NSUB + sid

        pltpu.sync_copy(src2d.at[pl.ds(wid * nch, nch)], srcv)
        pltpu.sync_copy(dst2d.at[pl.ds(wid * nch, nch)], dstv)

        zv = jnp.zeros((16,), jnp.float32)

        def zrow(i, carry):
            for j in range(w // 16):
                bufs[0][0][i, pl.ds(j * 16, 16)] = zv
            return carry

        lax.fori_loop(0, _CH, zrow, 0)

        def zcopy(k, carry):
            pltpu.sync_copy(bufs[0][0],
                            acc.at[pl.ds(sid * rpt + k * _CH, _CH)])
            if with_cnt:
                pltpu.sync_copy(bufs[0][0],
                                cacc.at[pl.ds(sid * rpt + k * _CH, _CH)])
            return carry

        lax.fori_loop(0, rpt // _CH, zcopy, 0)

        if with_cnt:
            ov = jnp.ones((16,), jnp.float32)

            def orow(i, carry):
                for j in range(w // 16):
                    obuf[i, pl.ds(j * 16, 16)] = ov
                return carry

            lax.fori_loop(0, _CH, orow, 0)

        plsc.subcore_barrier()

        npair = ngrp // 2

        def wait_gathers(bset):
            for j in range(grp):
                pltpu.make_async_copy(table.at[srcv.at[0]], bset[j],
                                      gsem).wait()

        def fire_gathers(bset, base):
            for j in range(grp):
                pltpu.async_copy(table.at[srcv.at[base + j]], bset[j], gsem)

        def fire_scatters(bset, base):
            for j in range(grp):
                pltpu.async_copy(bset[j], acc.at[dstv.at[base + j]], ssem,
                                 add=True)
                if with_cnt:
                    pltpu.async_copy(obuf, cacc.at[dstv.at[base + j]], osem,
                                     add=True)

        def drain_scatters(bset):
            for j in range(grp):
                pltpu.make_async_copy(bset[j], acc.at[dstv.at[0]],
                                      ssem).wait()

        fire_gathers(bufs[0], 0)

        def pair(q, carry):
            a = 2 * q * grp
            wait_gathers(bufs[0])

            @pl.when(q > 0)
            def _d1():
                drain_scatters(bufs[1])

            fire_gathers(bufs[1], a + grp)
            fire_scatters(bufs[0], a)
            wait_gathers(bufs[1])

            @pl.when(q + 1 < npair)
            def _d0():
                drain_scatters(bufs[0])
                fire_gathers(bufs[0], a + 2 * grp)

            fire_scatters(bufs[1], a + grp)
            return carry

        lax.fori_loop(0, npair, pair, 0)
        drain_scatters(bufs[0])
        drain_scatters(bufs[1])
        if with_cnt:
            def odrain(k, carry):
                pltpu.make_async_copy(obuf, cacc.at[dstv.at[0]], osem).wait()
                return carry

            lax.fori_loop(0, nch, odrain, 0)
        plsc.subcore_barrier()

        def cout(k, carry):
            pltpu.sync_copy(acc.at[pl.ds(sid * rpt + k * _CH, _CH)],
                            bufs[0][0])
            pltpu.async_copy(bufs[0][0],
                             out.at[cid, pl.ds(sid * rpt + k * _CH, _CH)],
                             gsem).wait()
            if with_cnt:
                pltpu.sync_copy(cacc.at[pl.ds(sid * rpt + k * _CH, _CH)],
                                bufs[1][0])
                pltpu.async_copy(
                    bufs[1][0],
                    cout_hbm.at[cid, pl.ds(sid * rpt + k * _CH, _CH)],
                    gsem).wait()
            return carry

        lax.fori_loop(0, rpt // _CH, cout, 0)

    return seg


def kernel(x, edge_index, Wl1, bl1, Wr1, g1, b1, Wl2, bl2, Wr2, g2, b2,
           Wl3, bl3, Wr3):
    n, d_in = x.shape
    e = edge_index.shape[1]
    d_h = Wl1.shape[0]
    d_out = Wl3.shape[0]
    f32 = jnp.float32

    pk = 128 // d_h
    n_pad = _pad_up(n + 1, 4 * _NSUB * _CH // d_h)
    e_pad = _pad_up(e, _NWORK * _CH * 10)
    nch = e_pad // (_NWORK * _CH)

    src = edge_index[0]
    dst = edge_index[1]
    pad = e_pad - e
    if pad:
        ar = jnp.arange(pad, dtype=jnp.int32)
        src = jnp.concatenate([src, ar % n])
        dst = jnp.concatenate([dst, n + ar % (n_pad - n)])
    src2d = src.reshape(e_pad // _CH, _CH)
    dst2d = dst.reshape(e_pad // _CH, _CH)

    eye = jnp.eye(pk, dtype=f32)
    wl1b = jnp.kron(eye, Wl1.T)
    wr1b = jnp.kron(eye, Wr1.T)
    wl2b = jnp.kron(eye, Wl2.T)
    wr2b = jnp.kron(eye, Wr2.T)
    w3p = jnp.zeros((d_h, d_h), f32).at[:, :d_out].set(Wl3.T)
    w3rp = jnp.zeros((d_h, d_h), f32).at[:, :d_out].set(Wr3.T)
    wl3b = jnp.kron(eye, w3p)
    wr3b = jnp.kron(eye, w3rp)
    mavg = jnp.kron(eye, jnp.full((d_h, d_h), 1.0 / d_h, f32))
    bl1p = jnp.tile(bl1, pk).reshape(1, 128)
    g1p = jnp.tile(g1, pk).reshape(1, 128)
    b1p = jnp.tile(b1, pk).reshape(1, 128)
    bl2p = jnp.tile(bl2, pk).reshape(1, 128)
    g2p = jnp.tile(g2, pk).reshape(1, 128)
    b2p = jnp.tile(b2, pk).reshape(1, 128)
    bl3p = jnp.tile(jnp.zeros((d_h,), f32).at[:d_out].set(bl3),
                    pk).reshape(1, 128)

    npk = n // pk
    rbp = npk
    grid = (1,)
    rs = lambda: pl.BlockSpec((npk, 128), lambda i: (0, 0))
    fs = lambda r: pl.BlockSpec((r, 128), lambda i: (0, 0))

    x4 = x.reshape(npk, pk * d_in)

    def tc1(x_ref, wl_ref, wr_ref, tab_ref, r_ref):
        xb = x_ref[:]
        tab_ref[:] = jnp.dot(xb, wl_ref[:], preferred_element_type=f32)
        r_ref[:] = jnp.dot(xb, wr_ref[:], preferred_element_type=f32)

    table1p, r1p = pl.pallas_call(
        tc1,
        grid=grid,
        in_specs=[pl.BlockSpec((npk, pk * d_in), lambda i: (0, 0)),
                  pl.BlockSpec((pk * d_in, 128), lambda i: (0, 0)),
                  pl.BlockSpec((pk * d_in, 128), lambda i: (0, 0))],
        out_specs=[rs(), rs()],
        out_shape=[jax.ShapeDtypeStruct((npk, 128), f32),
                   jax.ShapeDtypeStruct((npk, 128), f32)],
    )(x4, wl1b, wr1b)

    seg1 = _make_segsum(n_pad, d_h, nch, with_cnt=True)
    acc1, cnt1 = seg1(table1p.reshape(n, d_h), src2d, dst2d)
    acc1p = acc1.reshape(_NCORE, n_pad // pk, 128)
    cnt1p = cnt1.reshape(_NCORE, n_pad // pk, 128)

    def tc2(a0_ref, a1_ref, c0_ref, c1_ref, r_ref, bl_ref, g_ref, b_ref,
            mavg_ref, wl_ref, wr_ref, tab_ref, rn_ref, inv_ref):
        s = a0_ref[:] + a1_ref[:]
        inv = 1.0 / jnp.clip(c0_ref[:] + c1_ref[:], 1.0, None)
        pre = s * inv + bl_ref[:] + r_ref[:]
        mu = jnp.dot(pre, mavg_ref[:], preferred_element_type=f32)
        d = pre - mu
        var = jnp.dot(d * d, mavg_ref[:], preferred_element_type=f32)
        h = d / jnp.sqrt(var + 1e-5) * g_ref[:] + b_ref[:]
        h = jnp.maximum(h, 0.0)
        tab_ref[:] = jnp.dot(h, wl_ref[:], preferred_element_type=f32)
        rn_ref[:] = jnp.dot(h, wr_ref[:], preferred_element_type=f32)
        inv_ref[:] = inv

    table2p, r2p, invp = pl.pallas_call(
        tc2,
        grid=grid,
        in_specs=[rs(), rs(), rs(), rs(), rs(),
                  fs(1), fs(1), fs(1), fs(128), fs(128), fs(128)],
        out_specs=[rs(), rs(), rs()],
        out_shape=[jax.ShapeDtypeStruct((npk, 128), f32),
                   jax.ShapeDtypeStruct((npk, 128), f32),
                   jax.ShapeDtypeStruct((npk, 128), f32)],
    )(acc1p[0, :npk], acc1p[1, :npk], cnt1p[0, :npk], cnt1p[1, :npk], r1p,
      bl1p, g1p, b1p, mavg, wl2b, wr2b)

    seg2 = _make_segsum(n_pad, d_h, nch, with_cnt=False)
    acc2 = seg2(table2p.reshape(n, d_h), src2d, dst2d)
    acc2p = acc2.reshape(_NCORE, n_pad // pk, 128)

    def tc3(a0_ref, a1_ref, r_ref, inv_ref, bl_ref, g_ref, b_ref, mavg_ref,
            wl_ref, wr_ref, tab_ref, rn_ref):
        s = a0_ref[:] + a1_ref[:]
        pre = s * inv_ref[:] + bl_ref[:] + r_ref[:]
        mu = jnp.dot(pre, mavg_ref[:], preferred_element_type=f32)
        d = pre - mu
        var = jnp.dot(d * d, mavg_ref[:], preferred_element_type=f32)
        h = d / jnp.sqrt(var + 1e-5) * g_ref[:] + b_ref[:]
        h = jnp.maximum(h, 0.0)
        tab_ref[:] = jnp.dot(h, wl_ref[:], preferred_element_type=f32)
        rn_ref[:] = jnp.dot(h, wr_ref[:], preferred_element_type=f32)

    table3p, r3p = pl.pallas_call(
        tc3,
        grid=grid,
        in_specs=[rs(), rs(), rs(), rs(),
                  fs(1), fs(1), fs(1), fs(128), fs(128), fs(128)],
        out_specs=[rs(), rs()],
        out_shape=[jax.ShapeDtypeStruct((npk, 128), f32),
                   jax.ShapeDtypeStruct((npk, 128), f32)],
    )(acc2p[0, :npk], acc2p[1, :npk], r2p, invp,
      bl2p, g2p, b2p, mavg, wl3b, wr3b)

    seg3 = _make_segsum(n_pad, d_h, nch, with_cnt=False)
    acc3 = seg3(table3p.reshape(n, d_h), src2d, dst2d)
    acc3p = acc3.reshape(_NCORE, n_pad // pk, 128)

    def tc4(a0_ref, a1_ref, r_ref, inv_ref, bl_ref, out_ref):
        s = a0_ref[:] + a1_ref[:]
        out_ref[:] = s * inv_ref[:] + bl_ref[:] + r_ref[:]

    outp = pl.pallas_call(
        tc4,
        grid=grid,
        in_specs=[rs(), rs(), rs(), rs(), fs(1)],
        out_specs=rs(),
        out_shape=jax.ShapeDtypeStruct((npk, 128), f32),
    )(acc3p[0, :npk], acc3p[1, :npk], r3p, invp, bl3p)

    return outp.reshape(n, d_h)[:, :d_out]


# speedup vs baseline: 28.2933x; 1.0719x over previous
import functools

import jax
import jax.numpy as jnp
from jax import lax
from jax.experimental import pallas as pl
from jax.experimental.pallas import tpu as pltpu
from jax.experimental.pallas import tpu_sc as plsc

_CH = 128
_NCORE = 2
_NSUB = 16
_NWORK = _NCORE * _NSUB


def _pad_up(v, m):
    return (v + m - 1) // m * m


def _make_segsum(n_pad, w, nch, with_cnt):
    rpt = n_pad // _NSUB
    mesh = plsc.VectorSubcoreMesh(core_axis_name="c", subcore_axis_name="s")

    grp = 5 if with_cnt else 10
    assert nch % (2 * grp) == 0
    ngrp = nch // grp

    out_shape = jax.ShapeDtypeStruct((_NCORE, n_pad, w), jnp.float32)
    scratch = [
        pltpu.VMEM((nch, _CH), jnp.int32),
        pltpu.VMEM((nch, _CH), jnp.int32),
        *[pltpu.VMEM((_CH, w), jnp.float32) for _ in range(2 * grp)],
        pltpu.VMEM_SHARED((n_pad, w), jnp.float32),
        pltpu.SemaphoreType.DMA,
        pltpu.SemaphoreType.DMA,
    ]
    if with_cnt:
        scratch += [
            pltpu.VMEM((_CH, w), jnp.float32),
            pltpu.VMEM_SHARED((n_pad, w), jnp.float32),
            pltpu.SemaphoreType.DMA,
        ]

    @functools.partial(
        pl.kernel,
        out_type=[out_shape, out_shape] if with_cnt else out_shape,
        mesh=mesh,
        compiler_params=pltpu.CompilerParams(use_tc_tiling_on_sc=False),
        scratch_types=scratch,
    )
    def seg(table, src2d, dst2d, *rest):
        if with_cnt:
            out, cout_hbm = rest[0], rest[1]
            rest = rest[2:]
        else:
            out = rest[0]
            rest = rest[1:]
        srcv, dstv = rest[0], rest[1]
        bufs = (rest[2:2 + grp], rest[2 + grp:2 + 2 * grp])
        acc = rest[2 + 2 * grp]
        gsem, ssem = rest[3 + 2 * grp], rest[4 + 2 * grp]
        if with_cnt:
            obuf, cacc, osem = (rest[5 + 2 * grp], rest[6 + 2 * grp],
                                rest[7 + 2 * grp])
        cid = lax.axis_index("c")
        sid = lax.axis_index("s")
        wid = cid * _NSUB + sid

        pltpu.sync_copy(src2d.at[pl.ds(wid * nch, nch)], srcv)
        pltpu.sync_copy(dst2d.at[pl.ds(wid * nch, nch)], dstv)

        zv = jnp.zeros((16,), jnp.float32)

        def zrow(i, carry):
            for j in range(w // 16):
                bufs[0][0][i, pl.ds(j * 16, 16)] = zv
            return carry

        lax.fori_loop(0, _CH, zrow, 0)

        def zcopy(k, carry):
            pltpu.sync_copy(bufs[0][0],
                            acc.at[pl.ds(sid * rpt + k * _CH, _CH)])
            if with_cnt:
                pltpu.sync_copy(bufs[0][0],
                                cacc.at[pl.ds(sid * rpt + k * _CH, _CH)])
            return carry

        lax.fori_loop(0, rpt // _CH, zcopy, 0)

        if with_cnt:
            ov = jnp.ones((16,), jnp.float32)

            def orow(i, carry):
                for j in range(w // 16):
                    obuf[i, pl.ds(j * 16, 16)] = ov
                return carry

            lax.fori_loop(0, _CH, orow, 0)

        plsc.subcore_barrier()

        npair = ngrp // 2

        def wait_gathers(bset):
            for j in range(grp):
                pltpu.make_async_copy(table.at[srcv.at[0]], bset[j],
                                      gsem).wait()

        def fire_gathers(bset, base):
            for j in range(grp):
                pltpu.async_copy(table.at[srcv.at[base + j]], bset[j], gsem)

        def fire_scatters(bset, base):
            for j in range(grp):
                pltpu.async_copy(bset[j], acc.at[dstv.at[base + j]], ssem,
                                 add=True)
                if with_cnt:
                    pltpu.async_copy(obuf, cacc.at[dstv.at[base + j]], osem,
                                     add=True)

        def drain_scatters(bset):
            for j in range(grp):
                pltpu.make_async_copy(bset[j], acc.at[dstv.at[0]],
                                      ssem).wait()

        fire_gathers(bufs[0], 0)

        def pair(q, carry):
            a = 2 * q * grp
            wait_gathers(bufs[0])

            @pl.when(q > 0)
            def _d1():
                drain_scatters(bufs[1])

            fire_gathers(bufs[1], a + grp)
            fire_scatters(bufs[0], a)
            wait_gathers(bufs[1])

            @pl.when(q + 1 < npair)
            def _d0():
                drain_scatters(bufs[0])
                fire_gathers(bufs[0], a + 2 * grp)

            fire_scatters(bufs[1], a + grp)
            return carry

        lax.fori_loop(0, npair, pair, 0)
        drain_scatters(bufs[0])
        drain_scatters(bufs[1])
        if with_cnt:
            def odrain(k, carry):
                pltpu.make_async_copy(obuf, cacc.at[dstv.at[0]], osem).wait()
                return carry

            lax.fori_loop(0, nch, odrain, 0)
        plsc.subcore_barrier()

        def cout(k, carry):
            pltpu.sync_copy(acc.at[pl.ds(sid * rpt + k * _CH, _CH)],
                            bufs[0][0])
            pltpu.async_copy(bufs[0][0],
                             out.at[cid, pl.ds(sid * rpt + k * _CH, _CH)],
                             gsem).wait()
            if with_cnt:
                pltpu.sync_copy(cacc.at[pl.ds(sid * rpt + k * _CH, _CH)],
                                bufs[1][0])
                pltpu.async_copy(
                    bufs[1][0],
                    cout_hbm.at[cid, pl.ds(sid * rpt + k * _CH, _CH)],
                    gsem).wait()
            return carry

        lax.fori_loop(0, rpt // _CH, cout, 0)

    return seg


def kernel(x, edge_index, Wl1, bl1, Wr1, g1, b1, Wl2, bl2, Wr2, g2, b2,
           Wl3, bl3, Wr3):
    n, d_in = x.shape
    e = edge_index.shape[1]
    d_h = Wl1.shape[0]
    d_out = Wl3.shape[0]
    f32 = jnp.float32

    pk = 128 // d_h
    n_pad = _pad_up(n + 1, 4 * _NSUB * _CH // d_h)
    e_pad = _pad_up(e, _NWORK * _CH * 10)
    nch = e_pad // (_NWORK * _CH)

    src = edge_index[0]
    dst = edge_index[1]
    pad = e_pad - e
    if pad:
        ar = jnp.arange(pad, dtype=jnp.int32)
        src = jnp.concatenate([src, ar % n])
        dst = jnp.concatenate([dst, n + ar % (n_pad - n)])
    src2d = src.reshape(e_pad // _CH, _CH)
    dst2d = dst.reshape(e_pad // _CH, _CH)

    eye = jnp.eye(pk, dtype=f32)
    wl1b = jnp.kron(eye, Wl1.T)
    wr1b = jnp.kron(eye, Wr1.T)
    wl2b = jnp.kron(eye, Wl2.T)
    wr2b = jnp.kron(eye, Wr2.T)
    w3p = jnp.zeros((d_h, d_h), f32).at[:, :d_out].set(Wl3.T)
    w3rp = jnp.zeros((d_h, d_h), f32).at[:, :d_out].set(Wr3.T)
    wl3b = jnp.kron(eye, w3p)
    wr3b = jnp.kron(eye, w3rp)
    mavg = jnp.kron(eye, jnp.full((d_h, d_h), 1.0 / d_h, f32))
    bl1p = jnp.tile(bl1, pk).reshape(1, 128)
    g1p = jnp.tile(g1, pk).reshape(1, 128)
    b1p = jnp.tile(b1, pk).reshape(1, 128)
    bl2p = jnp.tile(bl2, pk).reshape(1, 128)
    g2p = jnp.tile(g2, pk).reshape(1, 128)
    b2p = jnp.tile(b2, pk).reshape(1, 128)
    bl3p = jnp.tile(jnp.zeros((d_h,), f32).at[:d_out].set(bl3),
                    pk).reshape(1, 128)

    npk = n // pk
    rbp = npk
    grid = (1,)
    rs = lambda: pl.BlockSpec((npk, 128), lambda i: (0, 0))
    a3 = lambda: pl.BlockSpec((_NCORE, n_pad // pk, 128), lambda i: (0, 0, 0))
    fs = lambda r: pl.BlockSpec((r, 128), lambda i: (0, 0))

    x4 = x.reshape(npk, pk * d_in)

    def tc1(x_ref, wl_ref, wr_ref, tab_ref, r_ref):
        xb = x_ref[:]
        tab_ref[:] = jnp.dot(xb, wl_ref[:], preferred_element_type=f32)
        r_ref[:] = jnp.dot(xb, wr_ref[:], preferred_element_type=f32)

    table1p, r1p = pl.pallas_call(
        tc1,
        grid=grid,
        in_specs=[pl.BlockSpec((npk, pk * d_in), lambda i: (0, 0)),
                  pl.BlockSpec((pk * d_in, 128), lambda i: (0, 0)),
                  pl.BlockSpec((pk * d_in, 128), lambda i: (0, 0))],
        out_specs=[rs(), rs()],
        out_shape=[jax.ShapeDtypeStruct((npk, 128), f32),
                   jax.ShapeDtypeStruct((npk, 128), f32)],
    )(x4, wl1b, wr1b)

    seg1 = _make_segsum(n_pad, d_h, nch, with_cnt=True)
    acc1, cnt1 = seg1(table1p.reshape(n, d_h), src2d, dst2d)
    acc1p = acc1.reshape(_NCORE, n_pad // pk, 128)
    cnt1p = cnt1.reshape(_NCORE, n_pad // pk, 128)

    def tc2(a_ref, c_ref, r_ref, bl_ref, g_ref, b_ref,
            mavg_ref, wl_ref, wr_ref, tab_ref, rn_ref, inv_ref):
        s = a_ref[0, :npk, :] + a_ref[1, :npk, :]
        inv = 1.0 / jnp.clip(c_ref[0, :npk, :] + c_ref[1, :npk, :], 1.0, None)
        pre = s * inv + bl_ref[:] + r_ref[:]
        mu = jnp.dot(pre, mavg_ref[:], preferred_element_type=f32)
        d = pre - mu
        var = jnp.dot(d * d, mavg_ref[:], preferred_element_type=f32)
        h = d / jnp.sqrt(var + 1e-5) * g_ref[:] + b_ref[:]
        h = jnp.maximum(h, 0.0)
        tab_ref[:] = jnp.dot(h, wl_ref[:], preferred_element_type=f32)
        rn_ref[:] = jnp.dot(h, wr_ref[:], preferred_element_type=f32)
        inv_ref[:] = inv

    table2p, r2p, invp = pl.pallas_call(
        tc2,
        grid=grid,
        in_specs=[a3(), a3(), rs(),
                  fs(1), fs(1), fs(1), fs(128), fs(128), fs(128)],
        out_specs=[rs(), rs(), rs()],
        out_shape=[jax.ShapeDtypeStruct((npk, 128), f32),
                   jax.ShapeDtypeStruct((npk, 128), f32),
                   jax.ShapeDtypeStruct((npk, 128), f32)],
    )(acc1p, cnt1p, r1p, bl1p, g1p, b1p, mavg, wl2b, wr2b)

    seg2 = _make_segsum(n_pad, d_h, nch, with_cnt=False)
    acc2 = seg2(table2p.reshape(n, d_h), src2d, dst2d)
    acc2p = acc2.reshape(_NCORE, n_pad // pk, 128)

    def tc3(a_ref, r_ref, inv_ref, bl_ref, g_ref, b_ref, mavg_ref,
            wl_ref, wr_ref, tab_ref, rn_ref):
        s = a_ref[0, :npk, :] + a_ref[1, :npk, :]
        pre = s * inv_ref[:] + bl_ref[:] + r_ref[:]
        mu = jnp.dot(pre, mavg_ref[:], preferred_element_type=f32)
        d = pre - mu
        var = jnp.dot(d * d, mavg_ref[:], preferred_element_type=f32)
        h = d / jnp.sqrt(var + 1e-5) * g_ref[:] + b_ref[:]
        h = jnp.maximum(h, 0.0)
        tab_ref[:] = jnp.dot(h, wl_ref[:], preferred_element_type=f32)
        rn_ref[:] = jnp.dot(h, wr_ref[:], preferred_element_type=f32)

    table3p, r3p = pl.pallas_call(
        tc3,
        grid=grid,
        in_specs=[a3(), rs(), rs(),
                  fs(1), fs(1), fs(1), fs(128), fs(128), fs(128)],
        out_specs=[rs(), rs()],
        out_shape=[jax.ShapeDtypeStruct((npk, 128), f32),
                   jax.ShapeDtypeStruct((npk, 128), f32)],
    )(acc2p, r2p, invp, bl2p, g2p, b2p, mavg, wl3b, wr3b)

    seg3 = _make_segsum(n_pad, d_h, nch, with_cnt=False)
    acc3 = seg3(table3p.reshape(n, d_h), src2d, dst2d)
    acc3p = acc3.reshape(_NCORE, n_pad // pk, 128)

    def tc4(a_ref, r_ref, inv_ref, bl_ref, out_ref):
        s = a_ref[0, :npk, :] + a_ref[1, :npk, :]
        out_ref[:] = s * inv_ref[:] + bl_ref[:] + r_ref[:]

    outp = pl.pallas_call(
        tc4,
        grid=grid,
        in_specs=[a3(), rs(), rs(), fs(1)],
        out_specs=rs(),
        out_shape=jax.ShapeDtypeStruct((npk, 128), f32),
    )(acc3p, r3p, invp, bl3p)

    return outp.reshape(n, d_h)[:, :d_out]
